# Initial kernel scaffold; baseline (speedup 1.0000x reference)
import jax
import jax.numpy as jnp
from jax.experimental import pallas as pl


def kernel(pix_to_face, bary_coords, face_verts_colors):
    raise NotImplementedError("write your pallas kernel here")


# speedup vs baseline: 2.8579x; 2.8579x over previous
import jax
import jax.numpy as jnp
from jax import lax
from jax.experimental import pallas as pl
from jax.experimental.pallas import tpu as pltpu
from jax.experimental.pallas import tpu_sc as plsc

B, H, W = 8, 512, 512
HW = H * W
N = B * HW
F = 100000
D = 16
NW = 32
NPW = N // NW
CH = 1024
SUB = CH // 128
NCHUNK = NPW // CH
WPB = HW // NPW


def _sc_body(pix_hbm, bary_hbm, table_hbm, out_hbm,
             idx_v, g_v, bary_v, out0_v, out1_v, out2_v, gsem):
    outs = (out0_v, out1_v, out2_v)
    cid = lax.axis_index("c")
    sid = lax.axis_index("s")
    wid = sid * 2 + cid
    b = wid // WPB
    inoff = (wid % WPB) * NPW

    iota = lax.iota(jnp.int32, 16)
    iota3 = iota * 3

    def chunk_body(chunk, _):
        base = wid * NPW + chunk * CH
        row0 = pl.multiple_of(base // 128, 8)
        pltpu.sync_copy(pix_hbm.at[pl.ds(row0, SUB)], idx_v)
        copies = [
            pltpu.async_copy(table_hbm.at[idx_v.at[j]], g_v.at[j], gsem)
            for j in range(SUB)
        ]
        pltpu.sync_copy(
            bary_hbm.at[pl.ds(pl.multiple_of(base * 3, 3072), CH * 3)], bary_v)
        for cpy in copies:
            cpy.wait()

        for j in range(SUB):
            gj = g_v.at[j]
            for k in range(8):
                p0 = j * 128 + k * 16
                f = idx_v[j, pl.ds(k * 16, 16)]
                mask = f > 0
                r16 = iota + (k * 16)
                bw = [plsc.load_gather(bary_v, [iota3 + (p0 * 3 + v)])
                      for v in range(3)]
                for c in range(3):
                    g0 = plsc.load_gather(gj, [r16, jnp.full((16,), c, jnp.int32)])
                    g1 = plsc.load_gather(gj, [r16, jnp.full((16,), 3 + c, jnp.int32)])
                    g2 = plsc.load_gather(gj, [r16, jnp.full((16,), 6 + c, jnp.int32)])
                    acc = bw[0] * g0 + bw[1] * g1 + bw[2] * g2
                    outs[c][pl.ds(p0, 16)] = jnp.where(
                        mask, acc, jnp.zeros_like(acc))

        dst0 = b * (3 * HW) + inoff + chunk * CH
        for c in range(3):
            pltpu.sync_copy(
                outs[c],
                out_hbm.at[pl.ds(pl.multiple_of(dst0 + c * HW, CH), CH)])
        return ()

    lax.fori_loop(0, NCHUNK, chunk_body, (), unroll=False)


@jax.jit
def _texture_shade(pix2d, bary_flat, table16):
    mesh = plsc.VectorSubcoreMesh(core_axis_name="c", subcore_axis_name="s")
    k = pl.kernel(
        _sc_body,
        out_type=jax.ShapeDtypeStruct((B * 3 * HW,), jnp.float32),
        mesh=mesh,
        compiler_params=pltpu.CompilerParams(
            needs_layout_passes=False, use_tc_tiling_on_sc=False),
        scratch_types=[
            pltpu.VMEM((SUB, 128), jnp.int32),
            pltpu.VMEM((SUB, 128, D), jnp.float32),
            pltpu.VMEM((CH * 3,), jnp.float32),
            pltpu.VMEM((CH,), jnp.float32),
            pltpu.VMEM((CH,), jnp.float32),
            pltpu.VMEM((CH,), jnp.float32),
            pltpu.SemaphoreType.DMA,
        ],
    )
    return k(pix2d, bary_flat, table16)


def kernel(pix_to_face, bary_coords, face_verts_colors):
    pix2d = pix_to_face.astype(jnp.int32).reshape(N // 128, 128)
    bary_flat = bary_coords.reshape(N * 3)
    table16 = jnp.pad(
        face_verts_colors.reshape(F, 9), ((0, 0), (0, D - 9)))
    out = _texture_shade(pix2d, bary_flat, table16)
    return out.reshape(B, 3, H, W)


# speedup vs baseline: 37.9397x; 13.2755x over previous
import jax
import jax.numpy as jnp
from jax import lax
from jax.experimental import pallas as pl
from jax.experimental.pallas import tpu as pltpu
from jax.experimental.pallas import tpu_sc as plsc

B, H, W = 8, 512, 512
HW = H * W
N = B * HW
F = 100000
D = 16
NW = 32
NPW = N // NW
CH = 1024
RPC = CH // W
SUB = CH // 128
NCHUNK = NPW // CH
WPB = HW // NPW


def _sc_body(pix_hbm, bary_hbm, table_hbm, out_hbm,
             idx_v, g_v, bary_v, out0_v, out1_v, out2_v, gsem):
    outs = (out0_v, out1_v, out2_v)
    cid = lax.axis_index("c")
    sid = lax.axis_index("s")
    wid = sid * 2 + cid
    b = wid // WPB
    inoff = (wid % WPB) * NPW

    iota = lax.iota(jnp.int32, 16)

    def chunk_body(chunk, _):
        base = wid * NPW + chunk * CH
        row0 = pl.multiple_of(base // 128, 8)
        pltpu.sync_copy(pix_hbm.at[pl.ds(row0, SUB)], idx_v)
        copies = [
            pltpu.async_copy(table_hbm.at[idx_v.at[j]], g_v.at[j], gsem)
            for j in range(SUB)
        ]
        boff = pl.multiple_of((base // W) * (3 * W), RPC * 3 * W)
        pltpu.sync_copy(bary_hbm.at[pl.ds(boff, RPC * 3 * W)], bary_v)
        for cpy in copies:
            cpy.wait()

        for j in range(SUB):
            gj = g_v.at[j]
            for k in range(8):
                p0 = j * 128 + k * 16
                f = idx_v[j, pl.ds(k * 16, 16)]
                mask = f > 0
                r16 = iota + (k * 16)
                r, w0 = divmod(p0, W)
                bw = [bary_v[pl.ds(r * 3 * W + v * W + w0, 16)]
                      for v in range(3)]
                for c in range(3):
                    g0 = plsc.load_gather(gj, [r16, jnp.full((16,), c, jnp.int32)])
                    g1 = plsc.load_gather(gj, [r16, jnp.full((16,), 3 + c, jnp.int32)])
                    g2 = plsc.load_gather(gj, [r16, jnp.full((16,), 6 + c, jnp.int32)])
                    acc = bw[0] * g0 + bw[1] * g1 + bw[2] * g2
                    outs[c][pl.ds(p0, 16)] = jnp.where(
                        mask, acc, jnp.zeros_like(acc))

        dst0 = b * (3 * HW) + inoff + chunk * CH
        for c in range(3):
            pltpu.sync_copy(
                outs[c],
                out_hbm.at[pl.ds(pl.multiple_of(dst0 + c * HW, CH), CH)])
        return ()

    lax.fori_loop(0, NCHUNK, chunk_body, (), unroll=False)


@jax.jit
def _texture_shade(pix2d, bary_t, table16):
    mesh = plsc.VectorSubcoreMesh(core_axis_name="c", subcore_axis_name="s")
    k = pl.kernel(
        _sc_body,
        out_type=jax.ShapeDtypeStruct((B * 3 * HW,), jnp.float32),
        mesh=mesh,
        compiler_params=pltpu.CompilerParams(
            needs_layout_passes=False, use_tc_tiling_on_sc=False),
        scratch_types=[
            pltpu.VMEM((SUB, 128), jnp.int32),
            pltpu.VMEM((SUB, 128, D), jnp.float32),
            pltpu.VMEM((RPC * 3 * W,), jnp.float32),
            pltpu.VMEM((CH,), jnp.float32),
            pltpu.VMEM((CH,), jnp.float32),
            pltpu.VMEM((CH,), jnp.float32),
            pltpu.SemaphoreType.DMA,
        ],
    )
    return k(pix2d, bary_t, table16)


def kernel(pix_to_face, bary_coords, face_verts_colors):
    pix2d = pix_to_face.astype(jnp.int32).reshape(N // 128, 128)
    bary_t = bary_coords.transpose(0, 1, 4, 3, 2).reshape(N * 3)
    table16 = jnp.pad(
        face_verts_colors.reshape(F, 9), ((0, 0), (0, D - 9)))
    out = _texture_shade(pix2d, bary_t, table16)
    return out.reshape(B, 3, H, W)


# speedup vs baseline: 45.7028x; 1.2046x over previous
import jax
import jax.numpy as jnp
from jax import lax
from jax.experimental import pallas as pl
from jax.experimental.pallas import tpu as pltpu
from jax.experimental.pallas import tpu_sc as plsc

B, H, W = 8, 512, 512
HW = H * W
N = B * HW
F = 100000
D = 16
NW = 32
NPW = N // NW
CH = 1024
RPC = CH // W
BPC = RPC * 3 * W
SUB = CH // 128
NCHUNK = NPW // CH
WPB = HW // NPW


def _sc_body(pix_hbm, bary_hbm, table_hbm, out_hbm,
             i0, i1, g0, g1, b0, b1,
             o00, o01, o02, o10, o11, o12,
             gs0, gs1, is0, is1, bs0, bs1, os0, os1):
    I = (i0, i1); G = (g0, g1); BV = (b0, b1)
    O = ((o00, o01, o02), (o10, o11, o12))
    GS = (gs0, gs1); IS = (is0, is1); BS = (bs0, bs1); OS = (os0, os1)

    cid = lax.axis_index("c")
    sid = lax.axis_index("s")
    wid = sid * 2 + cid
    b = wid // WPB
    inoff = (wid % WPB) * NPW

    iota = lax.iota(jnp.int32, 16)

    def idx_start(t, s):
        base = wid * NPW + t * CH
        row0 = pl.multiple_of(base // 128, 8)
        pltpu.async_copy(pix_hbm.at[pl.ds(row0, SUB)], I[s], IS[s])

    def idx_wait(s):
        pltpu.make_async_copy(pix_hbm.at[pl.ds(0, SUB)], I[s], IS[s]).wait()

    def bary_start(t, s):
        boff = pl.multiple_of((wid * NPW + t * CH) // W * (3 * W), BPC)
        pltpu.async_copy(bary_hbm.at[pl.ds(boff, BPC)], BV[s], BS[s])

    def bary_wait(s):
        pltpu.make_async_copy(bary_hbm.at[pl.ds(0, BPC)], BV[s], BS[s]).wait()

    def gathers_start(s):
        for j in range(SUB):
            pltpu.async_copy(table_hbm.at[I[s].at[j]], G[s].at[j], GS[s])

    def gathers_wait(s):
        for j in range(SUB):
            pltpu.make_async_copy(
                table_hbm.at[I[s].at[j]], G[s].at[j], GS[s]).wait()

    def out_start(t, s):
        dst0 = b * (3 * HW) + inoff + t * CH
        for c in range(3):
            pltpu.async_copy(
                O[s][c],
                out_hbm.at[pl.ds(pl.multiple_of(dst0 + c * HW, CH), CH)],
                OS[s])

    def out_wait(s):
        for c in range(3):
            pltpu.make_async_copy(
                O[s][c], out_hbm.at[pl.ds(0, CH)], OS[s]).wait()

    def compute(s):
        for j in range(SUB):
            gj = G[s].at[j]
            for k in range(8):
                p0 = j * 128 + k * 16
                f = I[s][j, pl.ds(k * 16, 16)]
                mask = f > 0
                r16 = iota + (k * 16)
                r, w0 = divmod(p0, W)
                bw = [BV[s][pl.ds(r * 3 * W + v * W + w0, 16)]
                      for v in range(3)]
                for c in range(3):
                    gg = [plsc.load_gather(
                        gj, [r16, jnp.full((16,), 3 * v + c, jnp.int32)])
                        for v in range(3)]
                    acc = bw[0] * gg[0] + bw[1] * gg[1] + bw[2] * gg[2]
                    O[s][c][pl.ds(p0, 16)] = jnp.where(
                        mask, acc, jnp.zeros_like(acc))

    idx_start(0, 0)
    idx_wait(0)
    gathers_start(0)
    bary_start(0, 0)
    idx_start(1, 1)

    def body(t2, _):
        for par in range(2):
            s = par
            t = t2 * 2 + par
            nxt = s ^ 1

            @pl.when(t + 1 < NCHUNK)
            def _():
                idx_wait(nxt)
                gathers_start(nxt)
                bary_start(t + 1, nxt)

            gathers_wait(s)
            bary_wait(s)

            @pl.when(t >= 2)
            def _():
                out_wait(s)

            compute(s)
            out_start(t, s)

            @pl.when(t + 2 < NCHUNK)
            def _():
                idx_start(t + 2, s)
        return ()

    lax.fori_loop(0, NCHUNK // 2, body, (), unroll=False)
    out_wait(0)
    out_wait(1)


@jax.jit
def _texture_shade(pix2d, bary_t, table16):
    mesh = plsc.VectorSubcoreMesh(core_axis_name="c", subcore_axis_name="s")
    k = pl.kernel(
        _sc_body,
        out_type=jax.ShapeDtypeStruct((B * 3 * HW,), jnp.float32),
        mesh=mesh,
        compiler_params=pltpu.CompilerParams(
            needs_layout_passes=False, use_tc_tiling_on_sc=False),
        scratch_types=(
            [pltpu.VMEM((SUB, 128), jnp.int32)] * 2
            + [pltpu.VMEM((SUB, 128, D), jnp.float32)] * 2
            + [pltpu.VMEM((BPC,), jnp.float32)] * 2
            + [pltpu.VMEM((CH,), jnp.float32)] * 6
            + [pltpu.SemaphoreType.DMA] * 8
        ),
    )
    return k(pix2d, bary_t, table16)


def kernel(pix_to_face, bary_coords, face_verts_colors):
    pix2d = pix_to_face.astype(jnp.int32).reshape(N // 128, 128)
    bary_t = bary_coords.transpose(0, 1, 4, 3, 2).reshape(N * 3)
    table16 = jnp.pad(
        face_verts_colors.reshape(F, 9), ((0, 0), (0, D - 9)))
    out = _texture_shade(pix2d, bary_t, table16)
    return out.reshape(B, 3, H, W)


# speedup vs baseline: 48.1022x; 1.0525x over previous
import jax
import jax.numpy as jnp
from jax import lax
from jax.experimental import pallas as pl
from jax.experimental.pallas import tpu as pltpu
from jax.experimental.pallas import tpu_sc as plsc

B, H, W = 8, 512, 512
HW = H * W
N = B * HW
F = 100000
D = 16
NW = 32
NPW = N // NW
CH = 1024
RPC = CH // W
BPC = RPC * 3 * W
SUB = CH // 128
NCHUNK = NPW // CH
WPB = HW // NPW


def _sc_body(pix_hbm, bary_hbm, table_hbm, out_hbm,
             i0, i1, g0, g1, b0, b1,
             o00, o01, o02, o10, o11, o12,
             gs0, gs1, is0, is1, bs0, bs1, os0, os1):
    I = (i0, i1); G = (g0, g1); BV = (b0, b1)
    O = ((o00, o01, o02), (o10, o11, o12))
    GS = (gs0, gs1); IS = (is0, is1); BS = (bs0, bs1); OS = (os0, os1)

    cid = lax.axis_index("c")
    sid = lax.axis_index("s")
    wid = sid * 2 + cid
    b = wid // WPB
    inoff = (wid % WPB) * NPW

    iota = lax.iota(jnp.int32, 16)

    def idx_start(t, s):
        base = wid * NPW + t * CH
        row0 = pl.multiple_of(base // 128, 8)
        pltpu.async_copy(pix_hbm.at[pl.ds(row0, SUB)], I[s], IS[s])

    def idx_wait(s):
        pltpu.make_async_copy(pix_hbm.at[pl.ds(0, SUB)], I[s], IS[s]).wait()

    def bary_start(t, s):
        boff = pl.multiple_of((wid * NPW + t * CH) // W * (3 * W), BPC)
        pltpu.async_copy(bary_hbm.at[pl.ds(boff, BPC)], BV[s], BS[s])

    def bary_wait(s):
        pltpu.make_async_copy(bary_hbm.at[pl.ds(0, BPC)], BV[s], BS[s]).wait()

    def gathers_start(s):
        for j in range(SUB):
            pltpu.async_copy(table_hbm.at[I[s].at[j]], G[s].at[j], GS[s])

    def gathers_wait(s):
        for j in range(SUB):
            pltpu.make_async_copy(
                table_hbm.at[I[s].at[j]], G[s].at[j], GS[s]).wait()

    def out_start(t, s):
        dst0 = b * (3 * HW) + inoff + t * CH
        for c in range(3):
            pltpu.async_copy(
                O[s][c],
                out_hbm.at[pl.ds(pl.multiple_of(dst0 + c * HW, CH), CH)],
                OS[s])

    def out_wait(s):
        for c in range(3):
            pltpu.make_async_copy(
                O[s][c], out_hbm.at[pl.ds(0, CH)], OS[s]).wait()

    def compute(s):
        for j in range(SUB):
            for k in range(8):
                p0 = j * 128 + k * 16
                f = I[s][j, pl.ds(k * 16, 16)]
                mask = f > 0
                gjk = G[s].at[j, pl.ds(k * 16, 16)]
                r, w0 = divmod(p0, W)
                bw = [BV[s][pl.ds(r * 3 * W + v * W + w0, 16)]
                      for v in range(3)]
                for c in range(3):
                    gg = [plsc.load_gather(
                        gjk, [iota, jnp.full((16,), 3 * v + c, jnp.int32)])
                        for v in range(3)]
                    acc = bw[0] * gg[0] + bw[1] * gg[1] + bw[2] * gg[2]
                    O[s][c][pl.ds(p0, 16)] = jnp.where(
                        mask, acc, jnp.zeros_like(acc))

    idx_start(0, 0)
    idx_wait(0)
    gathers_start(0)
    bary_start(0, 0)
    idx_start(1, 1)

    def body(t2, _):
        for par in range(2):
            s = par
            t = t2 * 2 + par
            nxt = s ^ 1

            @pl.when(t + 1 < NCHUNK)
            def _():
                idx_wait(nxt)
                gathers_start(nxt)
                bary_start(t + 1, nxt)

            gathers_wait(s)
            bary_wait(s)

            @pl.when(t >= 2)
            def _():
                out_wait(s)

            compute(s)
            out_start(t, s)

            @pl.when(t + 2 < NCHUNK)
            def _():
                idx_start(t + 2, s)
        return ()

    lax.fori_loop(0, NCHUNK // 2, body, (), unroll=False)
    out_wait(0)
    out_wait(1)


@jax.jit
def _texture_shade(pix2d, bary_t, table16):
    mesh = plsc.VectorSubcoreMesh(core_axis_name="c", subcore_axis_name="s")
    k = pl.kernel(
        _sc_body,
        out_type=jax.ShapeDtypeStruct((B * 3 * HW,), jnp.float32),
        mesh=mesh,
        compiler_params=pltpu.CompilerParams(
            needs_layout_passes=False, use_tc_tiling_on_sc=False),
        scratch_types=(
            [pltpu.VMEM((SUB, 128), jnp.int32)] * 2
            + [pltpu.VMEM((SUB, 128, D), jnp.float32)] * 2
            + [pltpu.VMEM((BPC,), jnp.float32)] * 2
            + [pltpu.VMEM((CH,), jnp.float32)] * 6
            + [pltpu.SemaphoreType.DMA] * 8
        ),
    )
    return k(pix2d, bary_t, table16)


def kernel(pix_to_face, bary_coords, face_verts_colors):
    pix2d = pix_to_face.astype(jnp.int32).reshape(N // 128, 128)
    bary_t = bary_coords.transpose(0, 1, 4, 3, 2).reshape(N * 3)
    table16 = jnp.pad(
        face_verts_colors.reshape(F, 9), ((0, 0), (0, D - 9)))
    out = _texture_shade(pix2d, bary_t, table16)
    return out.reshape(B, 3, H, W)
